# Initial kernel scaffold; baseline (speedup 1.0000x reference)
import jax
import jax.numpy as jnp
from jax.experimental import pallas as pl


def kernel(raw_triples, entity_map, relation_map):
    raise NotImplementedError("write your pallas kernel here")


# speedup vs baseline: 3.8960x; 3.8960x over previous
import functools

import jax
import jax.numpy as jnp
from jax import lax
from jax.experimental import pallas as pl
from jax.experimental.pallas import tpu as pltpu
from jax.experimental.pallas import tpu_sc as plsc

N_TRI = 1048576
N_ENT = 1000000
FLAT = 3 * N_TRI

NC = 2
NS = 16
NW = NC * NS
PER_W = FLAT // NW
CHUNK = 12288
STEPS = PER_W // CHUNK


def _make_sc_gather():
    mesh = plsc.VectorSubcoreMesh(core_axis_name="c", subcore_axis_name="s")

    @functools.partial(
        pl.kernel,
        mesh=mesh,
        out_type=jax.ShapeDtypeStruct((FLAT,), jnp.int32),
        scratch_types=[
            pltpu.VMEM((CHUNK,), jnp.int32),
            pltpu.VMEM((CHUNK,), jnp.int32),
            pltpu.SemaphoreType.DMA,
        ],
    )
    def body(flat_hbm, table_hbm, out_hbm, idx_v, val_v, sem):
        wid = lax.axis_index("s") * NC + lax.axis_index("c")
        base = wid * PER_W
        lanes = lax.iota(jnp.int32, 16)
        offs = [
            jnp.where((lanes + p) % 3 == 1, N_ENT, 0).astype(jnp.int32)
            for p in range(3)
        ]

        def step(i, _):
            off = base + i * CHUNK
            pltpu.sync_copy(flat_hbm.at[pl.ds(off, CHUNK)], idx_v)

            def fix(j, _):
                for p in range(3):
                    s = (j * 3 + p) * 16
                    idx_v[pl.ds(s, 16)] = idx_v[pl.ds(s, 16)] + offs[p]
                return 0

            lax.fori_loop(0, CHUNK // 48, fix, 0)
            pltpu.async_copy(table_hbm.at[idx_v], val_v, sem).wait()
            pltpu.sync_copy(val_v, out_hbm.at[pl.ds(off, CHUNK)])
            return 0

        lax.fori_loop(0, STEPS, step, 0)

    return body


_sc_gather = _make_sc_gather()


def kernel(raw_triples, entity_map, relation_map):
    table = jnp.concatenate(
        [entity_map.astype(jnp.int32), relation_map.astype(jnp.int32)]
    )
    flat = raw_triples.astype(jnp.int32).reshape(-1)
    out = _sc_gather(flat, table)
    return out.reshape(N_TRI, 3)


# speedup vs baseline: 5.0444x; 1.2948x over previous
import functools

import jax
import jax.numpy as jnp
from jax import lax
from jax.experimental import pallas as pl
from jax.experimental.pallas import tpu as pltpu
from jax.experimental.pallas import tpu_sc as plsc

N_TRI = 1048576
N_ENT = 1000000
FLAT = 3 * N_TRI

NC = 2
NS = 16
NW = NC * NS
PER_W = FLAT // NW
CHUNK = 12288
STEPS = PER_W // CHUNK

TAB = N_ENT + 1000
TAB_PAD = 1001472
STRIPE = TAB_PAD // NS


def _make_sc_gather():
    mesh = plsc.VectorSubcoreMesh(core_axis_name="c", subcore_axis_name="s")

    @functools.partial(
        pl.kernel,
        mesh=mesh,
        out_type=jax.ShapeDtypeStruct((FLAT,), jnp.int32),
        scratch_types=[
            pltpu.VMEM((CHUNK,), jnp.int32),
            pltpu.VMEM((CHUNK,), jnp.int32),
            pltpu.VMEM_SHARED((TAB_PAD,), jnp.int32),
            pltpu.SemaphoreType.DMA,
        ],
    )
    def body(flat_hbm, table_hbm, out_hbm, idx_v, val_v, tab_sh, sem):
        cid = lax.axis_index("c")
        sid = lax.axis_index("s")
        wid = sid * NC + cid
        base = wid * PER_W

        so = sid * STRIPE
        pltpu.sync_copy(table_hbm.at[pl.ds(so, STRIPE)], tab_sh.at[pl.ds(so, STRIPE)])
        plsc.subcore_barrier()

        lanes = lax.iota(jnp.int32, 16)
        offs = [
            jnp.where((lanes + p) % 3 == 1, N_ENT, 0).astype(jnp.int32)
            for p in range(3)
        ]

        def step(i, _):
            off = base + i * CHUNK
            pltpu.sync_copy(flat_hbm.at[pl.ds(off, CHUNK)], idx_v)

            def fix(j, _):
                for p in range(3):
                    s = (j * 3 + p) * 16
                    idx_v[pl.ds(s, 16)] = idx_v[pl.ds(s, 16)] + offs[p]
                return 0

            lax.fori_loop(0, CHUNK // 48, fix, 0)
            pltpu.async_copy(tab_sh.at[idx_v], val_v, sem).wait()
            pltpu.sync_copy(val_v, out_hbm.at[pl.ds(off, CHUNK)])
            return 0

        lax.fori_loop(0, STEPS, step, 0)

    return body


_sc_gather = _make_sc_gather()


def kernel(raw_triples, entity_map, relation_map):
    table = jnp.concatenate(
        [
            entity_map.astype(jnp.int32),
            relation_map.astype(jnp.int32),
            jnp.zeros((TAB_PAD - TAB,), jnp.int32),
        ]
    )
    flat = raw_triples.astype(jnp.int32).reshape(-1)
    out = _sc_gather(flat, table)
    return out.reshape(N_TRI, 3)


# speedup vs baseline: 84.1044x; 16.6729x over previous
import functools

import jax
import jax.numpy as jnp
from jax import lax
from jax.experimental import pallas as pl
from jax.experimental.pallas import tpu as pltpu
from jax.experimental.pallas import tpu_sc as plsc

N_TRI = 1048576
N_ENT = 1000000
N_REL = 1000

NC = 2
NS = 16
NW = NC * NS
PER_W = N_TRI // NW
CHUNK = 16384
STEPS = PER_W // CHUNK

TAB = N_ENT + N_REL
TAB_PAD = 1001472
STRIPE = TAB_PAD // NS


def _make_sc_gather():
    mesh = plsc.VectorSubcoreMesh(core_axis_name="c", subcore_axis_name="s")
    col = jax.ShapeDtypeStruct((N_TRI,), jnp.int32)

    @functools.partial(
        pl.kernel,
        mesh=mesh,
        out_type=(col, col, col),
        scratch_types=[
            pltpu.VMEM((CHUNK,), jnp.int32),
            pltpu.VMEM((CHUNK,), jnp.int32),
            pltpu.VMEM_SHARED((TAB_PAD,), jnp.int32),
            pltpu.SemaphoreType.DMA,
        ],
    )
    def body(h_hbm, r_hbm, t_hbm, tab_hbm,
             ho_hbm, ro_hbm, to_hbm, idx_v, val_v, tab_sh, sem):
        cid = lax.axis_index("c")
        sid = lax.axis_index("s")
        wid = sid * NC + cid
        base = wid * PER_W

        so = sid * STRIPE
        pltpu.sync_copy(tab_hbm.at[pl.ds(so, STRIPE)],
                        tab_sh.at[pl.ds(so, STRIPE)])
        plsc.subcore_barrier()

        def make_step(src_hbm, dst_hbm):
            def step(i, _):
                off = base + i * CHUNK
                pltpu.sync_copy(src_hbm.at[pl.ds(off, CHUNK)], idx_v)
                pltpu.async_copy(tab_sh.at[idx_v], val_v, sem).wait()
                pltpu.sync_copy(val_v, dst_hbm.at[pl.ds(off, CHUNK)])
                return 0
            return step

        lax.fori_loop(0, STEPS, make_step(h_hbm, ho_hbm), 0)
        lax.fori_loop(0, STEPS, make_step(r_hbm, ro_hbm), 0)
        lax.fori_loop(0, STEPS, make_step(t_hbm, to_hbm), 0)

    return body


_sc_gather = _make_sc_gather()


def kernel(raw_triples, entity_map, relation_map):
    raw_triples = raw_triples.astype(jnp.int32)
    h = raw_triples[:, 0]
    r = raw_triples[:, 1] + N_ENT
    t = raw_triples[:, 2]
    table = jnp.concatenate(
        [
            entity_map.astype(jnp.int32),
            relation_map.astype(jnp.int32),
            jnp.zeros((TAB_PAD - TAB,), jnp.int32),
        ]
    )
    h_enc, r_enc, t_enc = _sc_gather(h, r, t, table)
    return jnp.stack((h_enc, r_enc, t_enc), axis=1)


# speedup vs baseline: 91.7571x; 1.0910x over previous
import functools

import jax
import jax.numpy as jnp
from jax import lax
from jax.experimental import pallas as pl
from jax.experimental.pallas import tpu as pltpu
from jax.experimental.pallas import tpu_sc as plsc

N_TRI = 1048576
N_ENT = 1000000
N_REL = 1000

NC = 2
NS = 16
NW = NC * NS
PER_W = N_TRI // NW
CHUNK = 8192
STEPS = PER_W // CHUNK
NCHUNK = 3 * STEPS

TAB = N_ENT + N_REL
TAB_PAD = 1001472
STRIPE = TAB_PAD // NS


def _make_sc_gather():
    mesh = plsc.VectorSubcoreMesh(core_axis_name="c", subcore_axis_name="s")
    col = jax.ShapeDtypeStruct((N_TRI,), jnp.int32)

    @functools.partial(
        pl.kernel,
        mesh=mesh,
        out_type=(col, col, col),
        scratch_types=[
            pltpu.VMEM((CHUNK,), jnp.int32),
            pltpu.VMEM((CHUNK,), jnp.int32),
            pltpu.VMEM((CHUNK,), jnp.int32),
            pltpu.VMEM((CHUNK,), jnp.int32),
            pltpu.VMEM_SHARED((TAB_PAD,), jnp.int32),
            pltpu.SemaphoreType.DMA,
            pltpu.SemaphoreType.DMA,
            pltpu.SemaphoreType.DMA,
            pltpu.SemaphoreType.DMA,
            pltpu.SemaphoreType.DMA,
            pltpu.SemaphoreType.DMA,
        ],
    )
    def body(h_hbm, r_hbm, t_hbm, tab_hbm,
             ho_hbm, ro_hbm, to_hbm,
             idx0, idx1, val0, val1, tab_sh,
             si0, si1, sg0, sg1, so0, so1):
        cid = lax.axis_index("c")
        sid = lax.axis_index("s")
        wid = sid * NC + cid
        base = wid * PER_W

        idx = (idx0, idx1)
        val = (val0, val1)
        s_in = (si0, si1)
        s_g = (sg0, sg1)
        s_out = (so0, so1)

        srcs = (h_hbm, r_hbm, t_hbm)
        dsts = (ho_hbm, ro_hbm, to_hbm)

        def src_slice(k):
            s, i = divmod(k, STEPS)
            return srcs[s].at[pl.ds(base + i * CHUNK, CHUNK)]

        def dst_slice(k):
            s, i = divmod(k, STEPS)
            return dsts[s].at[pl.ds(base + i * CHUNK, CHUNK)]

        ins = [None] * (NCHUNK + 2)
        outs = [None] * NCHUNK
        gs = [None] * NCHUNK

        ins[0] = pltpu.async_copy(src_slice(0), idx[0], s_in[0])
        ins[1] = pltpu.async_copy(src_slice(1), idx[1], s_in[1])

        so = sid * STRIPE
        pltpu.sync_copy(tab_hbm.at[pl.ds(so, STRIPE)],
                        tab_sh.at[pl.ds(so, STRIPE)])
        plsc.subcore_barrier()

        ins[0].wait()
        gs[0] = pltpu.async_copy(tab_sh.at[idx[0]], val[0], s_g[0])

        for k in range(NCHUNK):
            b = k % 2
            nb = (k + 1) % 2
            if k + 1 < NCHUNK:
                ins[k + 1].wait()
                if k >= 1:
                    outs[k - 1].wait()
                gs[k + 1] = pltpu.async_copy(
                    tab_sh.at[idx[nb]], val[nb], s_g[nb])
            gs[k].wait()
            outs[k] = pltpu.async_copy(val[b], dst_slice(k), s_out[b])
            if k + 2 < NCHUNK:
                ins[k + 2] = pltpu.async_copy(src_slice(k + 2), idx[b], s_in[b])

        outs[NCHUNK - 2].wait()
        outs[NCHUNK - 1].wait()

    return body


_sc_gather = _make_sc_gather()


def kernel(raw_triples, entity_map, relation_map):
    raw_triples = raw_triples.astype(jnp.int32)
    h = raw_triples[:, 0]
    r = raw_triples[:, 1] + N_ENT
    t = raw_triples[:, 2]
    table = jnp.concatenate(
        [
            entity_map.astype(jnp.int32),
            relation_map.astype(jnp.int32),
            jnp.zeros((TAB_PAD - TAB,), jnp.int32),
        ]
    )
    h_enc, r_enc, t_enc = _sc_gather(h, r, t, table)
    return jnp.stack((h_enc, r_enc, t_enc), axis=1)


# speedup vs baseline: 110.8681x; 1.2083x over previous
import functools

import jax
import jax.numpy as jnp
from jax import lax
from jax.experimental import pallas as pl
from jax.experimental.pallas import tpu as pltpu
from jax.experimental.pallas import tpu_sc as plsc

N_TRI = 1048576
N_ENT = 1000000
N_REL = 1000

NC = 2
NS = 16
NW = NC * NS
PER_W = N_TRI // NW
CHUNK = 8192
STEPS = PER_W // CHUNK
NCHUNK = 3 * STEPS

ENT_STRIPE = 62464
ENT_BULK = NS * ENT_STRIPE
ENT2_PAD = 640
REL_OFF = 1000448
REL_PAD = 1024
TAB_PAD = REL_OFF + REL_PAD


def _make_sc_gather():
    mesh = plsc.VectorSubcoreMesh(core_axis_name="c", subcore_axis_name="s")
    col = jax.ShapeDtypeStruct((N_TRI,), jnp.int32)

    @functools.partial(
        pl.kernel,
        mesh=mesh,
        out_type=(col, col, col),
        scratch_types=[
            pltpu.VMEM((CHUNK,), jnp.int32),
            pltpu.VMEM((CHUNK,), jnp.int32),
            pltpu.VMEM((CHUNK,), jnp.int32),
            pltpu.VMEM((CHUNK,), jnp.int32),
            pltpu.VMEM_SHARED((TAB_PAD,), jnp.int32),
            pltpu.SemaphoreType.DMA,
            pltpu.SemaphoreType.DMA,
            pltpu.SemaphoreType.DMA,
            pltpu.SemaphoreType.DMA,
            pltpu.SemaphoreType.DMA,
            pltpu.SemaphoreType.DMA,
        ],
    )
    def body(h_hbm, r_hbm, t_hbm, ent_hbm, ent2_hbm, rel_hbm,
             ho_hbm, ro_hbm, to_hbm,
             idx0, idx1, val0, val1, tab_sh,
             si0, si1, sg0, sg1, so0, so1):
        cid = lax.axis_index("c")
        sid = lax.axis_index("s")
        wid = sid * NC + cid
        base = wid * PER_W

        idx = (idx0, idx1)
        val = (val0, val1)
        s_in = (si0, si1)
        s_g = (sg0, sg1)
        s_out = (so0, so1)

        srcs = (h_hbm, r_hbm, t_hbm)
        dsts = (ho_hbm, ro_hbm, to_hbm)

        def src_slice(k):
            s, i = divmod(k, STEPS)
            return srcs[s].at[pl.ds(base + i * CHUNK, CHUNK)]

        def dst_slice(k):
            s, i = divmod(k, STEPS)
            return dsts[s].at[pl.ds(base + i * CHUNK, CHUNK)]

        ins = [None] * (NCHUNK + 2)
        outs = [None] * NCHUNK
        gs = [None] * NCHUNK

        ins[0] = pltpu.async_copy(src_slice(0), idx[0], s_in[0])
        ins[1] = pltpu.async_copy(src_slice(1), idx[1], s_in[1])

        so = sid * ENT_STRIPE
        pltpu.sync_copy(ent_hbm.at[pl.ds(so, ENT_STRIPE)],
                        tab_sh.at[pl.ds(so, ENT_STRIPE)])

        @pl.when(sid == 0)
        def _():
            pltpu.sync_copy(rel_hbm, tab_sh.at[pl.ds(REL_OFF, REL_PAD)])

        @pl.when(sid == 1)
        def _():
            pltpu.sync_copy(ent2_hbm, tab_sh.at[pl.ds(ENT_BULK, ENT2_PAD)])

        plsc.subcore_barrier()

        ins[0].wait()
        gs[0] = pltpu.async_copy(tab_sh.at[idx[0]], val[0], s_g[0])

        for k in range(NCHUNK):
            b = k % 2
            nb = (k + 1) % 2
            if k + 1 < NCHUNK:
                ins[k + 1].wait()
                if k >= 1:
                    outs[k - 1].wait()
                gs[k + 1] = pltpu.async_copy(
                    tab_sh.at[idx[nb]], val[nb], s_g[nb])
            gs[k].wait()
            outs[k] = pltpu.async_copy(val[b], dst_slice(k), s_out[b])
            if k + 2 < NCHUNK:
                ins[k + 2] = pltpu.async_copy(src_slice(k + 2), idx[b], s_in[b])

        outs[NCHUNK - 2].wait()
        outs[NCHUNK - 1].wait()

    return body


_sc_gather = _make_sc_gather()


def kernel(raw_triples, entity_map, relation_map):
    raw_triples = raw_triples.astype(jnp.int32)
    h = raw_triples[:, 0]
    r = raw_triples[:, 1] + REL_OFF
    t = raw_triples[:, 2]
    ent = entity_map.astype(jnp.int32)
    ent2 = jnp.pad(ent[ENT_BULK:], (0, ENT2_PAD - (N_ENT - ENT_BULK)))
    rel = jnp.pad(relation_map.astype(jnp.int32), (0, REL_PAD - N_REL))
    h_enc, r_enc, t_enc = _sc_gather(h, r, t, ent, ent2, rel)
    return jnp.stack((h_enc, r_enc, t_enc), axis=1)


# speedup vs baseline: 121.4950x; 1.0959x over previous
import functools

import jax
import jax.numpy as jnp
from jax import lax
from jax.experimental import pallas as pl
from jax.experimental.pallas import tpu as pltpu
from jax.experimental.pallas import tpu_sc as plsc

N_TRI = 1048576
N_ENT = 1000000
N_REL = 1000

NC = 2
NS = 16
NW = NC * NS
PER_W = N_TRI // NW
CHUNK = 8192
STEPS = PER_W // CHUNK
NCHUNK = 2 * STEPS

ENT_STRIPE = 62464
ENT_BULK = NS * ENT_STRIPE
ENT2_PAD = 640
TAB_PAD = 1000448
REL_PAD = 1024

R_UNROLL = 4
HALF_R = PER_W // 2
R_SLICE = HALF_R // STEPS


def _make_sc_gather():
    mesh = plsc.VectorSubcoreMesh(core_axis_name="c", subcore_axis_name="s")
    col = jax.ShapeDtypeStruct((N_TRI,), jnp.int32)

    @functools.partial(
        pl.kernel,
        mesh=mesh,
        compiler_params=pltpu.CompilerParams(needs_layout_passes=False),
        out_type=(col, col, col),
        scratch_types=[
            pltpu.VMEM((CHUNK,), jnp.int32),
            pltpu.VMEM((CHUNK,), jnp.int32),
            pltpu.VMEM((CHUNK,), jnp.int32),
            pltpu.VMEM((CHUNK,), jnp.int32),
            pltpu.VMEM((HALF_R,), jnp.int32),
            pltpu.VMEM((HALF_R,), jnp.int32),
            pltpu.VMEM((REL_PAD,), jnp.int32),
            pltpu.VMEM_SHARED((TAB_PAD,), jnp.int32),
            pltpu.SemaphoreType.DMA,
            pltpu.SemaphoreType.DMA,
            pltpu.SemaphoreType.DMA,
            pltpu.SemaphoreType.DMA,
            pltpu.SemaphoreType.DMA,
            pltpu.SemaphoreType.DMA,
            pltpu.SemaphoreType.DMA,
            pltpu.SemaphoreType.DMA,
        ],
    )
    def body(h_hbm, r_hbm, t_hbm, ent_hbm, ent2_hbm, rel_hbm,
             ho_hbm, ro_hbm, to_hbm,
             idx0, idx1, val0, val1, ridx, rval, rel_v, tab_sh,
             si0, si1, sg0, sg1, so0, so1, srin, srout):
        cid = lax.axis_index("c")
        sid = lax.axis_index("s")
        wid = sid * NC + cid
        base = wid * PER_W

        idx = (idx0, idx1)
        val = (val0, val1)
        s_in = (si0, si1)
        s_g = (sg0, sg1)
        s_out = (so0, so1)

        srcs = (h_hbm, t_hbm)
        dsts = (ho_hbm, to_hbm)

        def src_slice(k):
            s, i = divmod(k, STEPS)
            return srcs[s].at[pl.ds(base + i * CHUNK, CHUNK)]

        def dst_slice(k):
            s, i = divmod(k, STEPS)
            return dsts[s].at[pl.ds(base + i * CHUNK, CHUNK)]

        ins = [None] * (NCHUNK + 2)
        outs = [None] * NCHUNK
        gs = [None] * NCHUNK

        ins[0] = pltpu.async_copy(src_slice(0), idx[0], s_in[0])
        ins[1] = pltpu.async_copy(src_slice(1), idx[1], s_in[1])
        r_in = pltpu.async_copy(r_hbm.at[pl.ds(base, HALF_R)], ridx, srin)

        so = sid * ENT_STRIPE
        pltpu.sync_copy(ent_hbm.at[pl.ds(so, ENT_STRIPE)],
                        tab_sh.at[pl.ds(so, ENT_STRIPE)])

        @pl.when(sid == 1)
        def _():
            pltpu.sync_copy(ent2_hbm, tab_sh.at[pl.ds(ENT_BULK, ENT2_PAD)])

        pltpu.sync_copy(rel_hbm, rel_v)
        plsc.subcore_barrier()

        ins[0].wait()
        gs[0] = pltpu.async_copy(tab_sh.at[idx[0]], val[0], s_g[0])
        r_in.wait()

        def r_slice(k):
            r0 = (k % STEPS) * R_SLICE

            def it(j, _):
                for u in range(R_UNROLL):
                    s = r0 + (j * R_UNROLL + u) * 16
                    v = ridx[pl.ds(s, 16)]
                    rval[pl.ds(s, 16)] = plsc.load_gather(rel_v, [v])
                return 0

            lax.fori_loop(0, R_SLICE // (16 * R_UNROLL), it, 0)

        r_out0 = None
        for k in range(NCHUNK):
            b = k % 2
            nb = (k + 1) % 2
            if k + 1 < NCHUNK:
                ins[k + 1].wait()
                if k >= 1:
                    outs[k - 1].wait()
                gs[k + 1] = pltpu.async_copy(
                    tab_sh.at[idx[nb]], val[nb], s_g[nb])
            if k == STEPS:
                r_in.wait()
                r_out0.wait()
            r_slice(k)
            if k == STEPS - 1:
                r_out0 = pltpu.async_copy(
                    rval, ro_hbm.at[pl.ds(base, HALF_R)], srout)
                r_in = pltpu.async_copy(
                    r_hbm.at[pl.ds(base + HALF_R, HALF_R)], ridx, srin)
            gs[k].wait()
            outs[k] = pltpu.async_copy(val[b], dst_slice(k), s_out[b])
            if k + 2 < NCHUNK:
                ins[k + 2] = pltpu.async_copy(src_slice(k + 2), idx[b], s_in[b])

        r_out1 = pltpu.async_copy(
            rval, ro_hbm.at[pl.ds(base + HALF_R, HALF_R)], srout)
        outs[NCHUNK - 2].wait()
        outs[NCHUNK - 1].wait()
        r_out1.wait()

    return body


_sc_gather = _make_sc_gather()


def kernel(raw_triples, entity_map, relation_map):
    raw_triples = raw_triples.astype(jnp.int32)
    h = raw_triples[:, 0]
    r = raw_triples[:, 1]
    t = raw_triples[:, 2]
    ent = entity_map.astype(jnp.int32)
    ent2 = jnp.pad(ent[ENT_BULK:], (0, ENT2_PAD - (N_ENT - ENT_BULK)))
    rel = jnp.pad(relation_map.astype(jnp.int32), (0, REL_PAD - N_REL))
    h_enc, r_enc, t_enc = _sc_gather(h, r, t, ent, ent2, rel)
    return jnp.stack((h_enc, r_enc, t_enc), axis=1)
